# baseline (speedup 1.0000x reference)
import functools

import jax
import jax.numpy as jnp
from jax import lax
from jax.experimental import pallas as pl
from jax.experimental.pallas import tpu as pltpu
from jax.experimental.pallas import tpu_sc as plsc

N = 10000
D = 256
HALF = 128
E = 160000
NUM_GRAPHS = 16
LAYERS = 3

NC = 2
NS = 16
CHUNK = 128
CH = 80
GRP = 16
EPT = CH * CHUNK
E_PAD = NS * CH * CHUNK
SLAB = 624
TAIL_OFF = NS * SLAB
TAIL = N - TAIL_OFF
ACC_ROWS = N + 8


def _chain_body(x_hbm, dst_hbm, src_hbm, x1_hbm, x2_hbm, out_hbm, dst_v,
                src_v, rows_a, rows_b, acc_sh, sem_ga, sem_gb, sem_sa,
                sem_sb, sem_i):
    c = lax.axis_index("c")
    s = lax.axis_index("s")
    base = s * SLAB
    w = c * NS + s

    pltpu.sync_copy(x_hbm.at[pl.ds(c * N + base, SLAB)],
                    acc_sh.at[pl.ds(base, SLAB)])

    @pl.when(s == NS - 1)
    def _():
        pltpu.sync_copy(x_hbm.at[pl.ds(c * N + TAIL_OFF, TAIL)],
                        acc_sh.at[pl.ds(TAIL_OFF, TAIL)])

    pltpu.sync_copy(dst_hbm.at[pl.ds(w * EPT, EPT)], dst_v)

    def load_grp(g):
        return pltpu.async_copy(
            src_hbm.at[s, pl.ds(g * GRP, GRP)],
            src_v.at[pl.ds((g % 2) * GRP, GRP)], sem_i)

    def wait_grp(g):
        pltpu.make_async_copy(
            src_hbm.at[s, pl.ds(g * GRP, GRP)],
            src_v.at[pl.ds((g % 2) * GRP, GRP)], sem_i).wait()

    def run_layer(src_x):
        def gather(j, buf, sem):
            return pltpu.async_copy(
                src_x.at[dst_v.at[pl.ds(j * CHUNK, CHUNK)]], buf, sem)

        def scat(j, buf, sem):
            return pltpu.async_copy(buf, acc_sh.at[src_v.at[j % (2 * GRP)]],
                                    sem, add=True)

        def wait_g(j, buf, sem):
            pltpu.make_async_copy(
                src_x.at[dst_v.at[pl.ds(j * CHUNK, CHUNK)]], buf, sem).wait()

        def wait_s(j, buf, sem):
            pltpu.make_async_copy(buf, acc_sh.at[src_v.at[j % (2 * GRP)]],
                                  sem).wait()

        pltpu.sync_copy(src_hbm.at[s, pl.ds(0, GRP)],
                        src_v.at[pl.ds(0, GRP)])
        load_grp(1)
        gather(0, rows_a, sem_ga)
        gather(1, rows_b, sem_gb)

        def body(i, carry):
            j0 = 2 * i
            j1 = j0 + 1
            at_grp = jnp.logical_and(j0 % GRP == 0, j0 > 0)

            @pl.when(at_grp)
            def _():
                wait_grp(0)

            @pl.when(jnp.logical_and(at_grp, j0 + GRP < CH))
            def _():
                load_grp(j0 // GRP + 1)

            wait_g(j0, rows_a, sem_ga)
            scat(j0, rows_a, sem_sa)
            wait_s(j0, rows_a, sem_sa)

            @pl.when(j0 + 2 < CH)
            def _():
                gather(j0 + 2, rows_a, sem_ga)

            wait_g(j1, rows_b, sem_gb)
            scat(j1, rows_b, sem_sb)
            wait_s(j1, rows_b, sem_sb)

            @pl.when(j1 + 2 < CH)
            def _():
                gather(j1 + 2, rows_b, sem_gb)

            return carry

        lax.fori_loop(0, CH // 2, body, 0)

    def writeback(dst_x):
        pltpu.sync_copy(acc_sh.at[pl.ds(base, SLAB)],
                        dst_x.at[pl.ds(c * N + base, SLAB)])

        @pl.when(s == NS - 1)
        def _():
            pltpu.sync_copy(acc_sh.at[pl.ds(TAIL_OFF, TAIL)],
                            dst_x.at[pl.ds(c * N + TAIL_OFF, TAIL)])

    plsc.subcore_barrier()
    run_layer(x_hbm)
    plsc.subcore_barrier()
    writeback(x1_hbm)
    plsc.subcore_barrier()
    run_layer(x1_hbm)
    plsc.subcore_barrier()
    writeback(x2_hbm)
    plsc.subcore_barrier()
    run_layer(x2_hbm)
    plsc.subcore_barrier()
    writeback(out_hbm)


@functools.lru_cache(maxsize=1)
def _make_chain():
    mesh = plsc.VectorSubcoreMesh(core_axis_name="c", subcore_axis_name="s",
                                  num_cores=NC, num_subcores=NS)
    return pl.kernel(
        _chain_body,
        out_type=[
            jax.ShapeDtypeStruct((NC * N, HALF), jnp.float32),
            jax.ShapeDtypeStruct((NC * N, HALF), jnp.float32),
            jax.ShapeDtypeStruct((NC * N, HALF), jnp.float32),
        ],
        mesh=mesh,
        scratch_types=[
            pltpu.VMEM((EPT,), jnp.int32),
            pltpu.VMEM((2 * GRP, CHUNK), jnp.int32),
            pltpu.VMEM((CHUNK, HALF), jnp.float32),
            pltpu.VMEM((CHUNK, HALF), jnp.float32),
            pltpu.VMEM_SHARED((ACC_ROWS, HALF), jnp.float32),
            pltpu.SemaphoreType.DMA,
            pltpu.SemaphoreType.DMA,
            pltpu.SemaphoreType.DMA,
            pltpu.SemaphoreType.DMA,
            pltpu.SemaphoreType.DMA,
        ],
    )


MB = 1000
GRID = N // MB


def _fw_merge_body(xf0, xf1, w4, b, part_ref):
    part_ref[...] = (
        jnp.dot(xf0[...], w4[0], preferred_element_type=jnp.float32)
        + jnp.dot(xf1[...], w4[1], preferred_element_type=jnp.float32)
        + b[...])


def _fw_merge_call(xf, w4, b):
    return pl.pallas_call(
        _fw_merge_body,
        grid=(GRID,),
        in_specs=[
            pl.BlockSpec((MB, HALF), lambda i: (i, 0)),
            pl.BlockSpec((MB, HALF), lambda i: (i + GRID, 0)),
            pl.BlockSpec((4, HALF, D), lambda i: (0, 0, 0)),
            pl.BlockSpec((1, D), lambda i: (0, 0)),
        ],
        out_specs=pl.BlockSpec((MB, D), lambda i: (i, 0)),
        out_shape=jax.ShapeDtypeStruct((N, D), jnp.float32),
        compiler_params=pltpu.CompilerParams(
            dimension_semantics=("arbitrary",)),
    )(xf, xf, w4, b)


def _merge_body(part, xb0, xb1, w4, bat, g_ref, out_ref):
    i = pl.program_id(0)
    ho = (part[...]
          + jnp.dot(xb0[...], w4[2], preferred_element_type=jnp.float32)
          + jnp.dot(xb1[...], w4[3], preferred_element_type=jnp.float32))
    out_ref[...] = ho

    @pl.when(i == 0)
    def _():
        g_ref[...] = jnp.full((NUM_GRAPHS, D), -jnp.inf, jnp.float32)

    batv = bat[...]
    for g in range(NUM_GRAPHS):
        m = jnp.where(batv == g, ho, -jnp.inf)
        part = jnp.max(m, axis=0, keepdims=True)
        g_ref[g:g + 1, :] = jnp.maximum(g_ref[g:g + 1, :], part)


def _merge_call(part, xb, w4, batb):
    return pl.pallas_call(
        _merge_body,
        grid=(GRID,),
        in_specs=[
            pl.BlockSpec((MB, D), lambda i: (i, 0)),
            pl.BlockSpec((MB, HALF), lambda i: (i, 0)),
            pl.BlockSpec((MB, HALF), lambda i: (i + GRID, 0)),
            pl.BlockSpec((4, HALF, D), lambda i: (0, 0, 0)),
            pl.BlockSpec((MB, D), lambda i: (i, 0)),
        ],
        out_specs=[
            pl.BlockSpec((NUM_GRAPHS, D), lambda i: (0, 0)),
            pl.BlockSpec((MB, D), lambda i: (i, 0)),
        ],
        out_shape=[
            jax.ShapeDtypeStruct((NUM_GRAPHS, D), jnp.float32),
            jax.ShapeDtypeStruct((N, D), jnp.float32),
        ],
        compiler_params=pltpu.CompilerParams(
            dimension_semantics=("arbitrary",)),
    )(part, xb, xb, w4, batb)


def _prep_edges(ei):
    src = ei[0]
    dst = ei[1]
    pad = E_PAD - E
    dstp = jnp.concatenate([dst, jnp.zeros((pad,), jnp.int32)])
    srcp = jnp.concatenate([src, jnp.full((pad,), N, jnp.int32)])
    dst2 = jnp.stack([dstp, dstp + N]).reshape(2 * NS * EPT)
    src2 = srcp.reshape(NS, CH, CHUNK)
    return dst2, src2


def kernel(h, fw_edge_index, bw_edge_index, batch, W_merge, b_merge):
    chain = _make_chain()
    x0 = h.reshape(N, 2, HALF).transpose(1, 0, 2).reshape(2 * N, HALF)
    fw_dst, fw_src = _prep_edges(fw_edge_index)
    bw_dst, bw_src = _prep_edges(bw_edge_index)
    w4 = W_merge.reshape(D, 4, HALF).transpose(1, 2, 0)
    batb = jnp.broadcast_to(batch[:, None], (N, D))
    xf = chain(x0, fw_dst, fw_src)[2]
    part = _fw_merge_call(xf, w4, b_merge.reshape(1, D))
    xb = chain(x0, bw_dst, bw_src)[2]
    g_h, h_out = _merge_call(part, xb, w4, batb)
    return (g_h, h_out)


# speedup vs baseline: 1.0547x; 1.0547x over previous
import functools

import jax
import jax.numpy as jnp
from jax import lax
from jax.experimental import pallas as pl
from jax.experimental.pallas import tpu as pltpu
from jax.experimental.pallas import tpu_sc as plsc

N = 10000
D = 256
HALF = 128
E = 160000
NUM_GRAPHS = 16
LAYERS = 3

NC = 2
NS = 16
CHUNK = 104
CH = 98
EPT = CH * CHUNK
E_PAD = NS * CH * CHUNK
SLAB = 624
TAIL_OFF = NS * SLAB
TAIL = N - TAIL_OFF
ACC_ROWS = N + 8


def _chain_body(x_hbm, dst_hbm, src_hbm, x1_hbm, x2_hbm, out_hbm, dst_v,
                src_v, rows_a, rows_b, acc_sh, sem_ga, sem_gb, sem_sa,
                sem_sb):
    c = lax.axis_index("c")
    s = lax.axis_index("s")
    base = s * SLAB
    w = c * NS + s

    pltpu.sync_copy(x_hbm.at[pl.ds(c * N + base, SLAB)],
                    acc_sh.at[pl.ds(base, SLAB)])

    @pl.when(s == NS - 1)
    def _():
        pltpu.sync_copy(x_hbm.at[pl.ds(c * N + TAIL_OFF, TAIL)],
                        acc_sh.at[pl.ds(TAIL_OFF, TAIL)])

    pltpu.sync_copy(dst_hbm.at[pl.ds(w * EPT, EPT)], dst_v)
    pltpu.sync_copy(src_hbm.at[s], src_v)

    def run_layer(src_x):
        def gather(j, buf, sem):
            return pltpu.async_copy(
                src_x.at[dst_v.at[pl.ds(j * CHUNK, CHUNK)]], buf, sem)

        def scat(j, buf, sem):
            return pltpu.async_copy(buf, acc_sh.at[src_v.at[j]], sem,
                                    add=True)

        def wait_g(j, buf, sem):
            pltpu.make_async_copy(
                src_x.at[dst_v.at[pl.ds(j * CHUNK, CHUNK)]], buf, sem).wait()

        def wait_s(j, buf, sem):
            pltpu.make_async_copy(buf, acc_sh.at[src_v.at[j]], sem).wait()

        gather(0, rows_a, sem_ga)
        gather(1, rows_b, sem_gb)

        def body(i, carry):
            j0 = 2 * i
            j1 = j0 + 1
            wait_g(j0, rows_a, sem_ga)
            scat(j0, rows_a, sem_sa)
            wait_s(j0, rows_a, sem_sa)

            @pl.when(j0 + 2 < CH)
            def _():
                gather(j0 + 2, rows_a, sem_ga)

            wait_g(j1, rows_b, sem_gb)
            scat(j1, rows_b, sem_sb)
            wait_s(j1, rows_b, sem_sb)

            @pl.when(j1 + 2 < CH)
            def _():
                gather(j1 + 2, rows_b, sem_gb)

            return carry

        lax.fori_loop(0, CH // 2, body, 0)

    def writeback(dst_x):
        pltpu.sync_copy(acc_sh.at[pl.ds(base, SLAB)],
                        dst_x.at[pl.ds(c * N + base, SLAB)])

        @pl.when(s == NS - 1)
        def _():
            pltpu.sync_copy(acc_sh.at[pl.ds(TAIL_OFF, TAIL)],
                            dst_x.at[pl.ds(c * N + TAIL_OFF, TAIL)])

    plsc.subcore_barrier()
    run_layer(x_hbm)
    plsc.subcore_barrier()
    writeback(x1_hbm)
    plsc.subcore_barrier()
    run_layer(x1_hbm)
    plsc.subcore_barrier()
    writeback(x2_hbm)
    plsc.subcore_barrier()
    run_layer(x2_hbm)
    plsc.subcore_barrier()
    writeback(out_hbm)


@functools.lru_cache(maxsize=1)
def _make_chain():
    mesh = plsc.VectorSubcoreMesh(core_axis_name="c", subcore_axis_name="s",
                                  num_cores=NC, num_subcores=NS)
    return pl.kernel(
        _chain_body,
        out_type=[
            jax.ShapeDtypeStruct((NC * N, HALF), jnp.float32),
            jax.ShapeDtypeStruct((NC * N, HALF), jnp.float32),
            jax.ShapeDtypeStruct((NC * N, HALF), jnp.float32),
        ],
        mesh=mesh,
        scratch_types=[
            pltpu.VMEM((EPT,), jnp.int32),
            pltpu.VMEM((CH, CHUNK), jnp.int32),
            pltpu.VMEM((CHUNK, HALF), jnp.float32),
            pltpu.VMEM((CHUNK, HALF), jnp.float32),
            pltpu.VMEM_SHARED((ACC_ROWS, HALF), jnp.float32),
            pltpu.SemaphoreType.DMA,
            pltpu.SemaphoreType.DMA,
            pltpu.SemaphoreType.DMA,
            pltpu.SemaphoreType.DMA,
        ],
    )


MB = 1000
GRID = N // MB


def _fw_merge_body(xf0, xf1, w4, b, part_ref):
    part_ref[...] = (
        jnp.dot(xf0[...], w4[0], preferred_element_type=jnp.float32)
        + jnp.dot(xf1[...], w4[1], preferred_element_type=jnp.float32)
        + b[...])


def _fw_merge_call(xf, w4, b):
    return pl.pallas_call(
        _fw_merge_body,
        grid=(GRID,),
        in_specs=[
            pl.BlockSpec((MB, HALF), lambda i: (i, 0)),
            pl.BlockSpec((MB, HALF), lambda i: (i + GRID, 0)),
            pl.BlockSpec((4, HALF, D), lambda i: (0, 0, 0)),
            pl.BlockSpec((1, D), lambda i: (0, 0)),
        ],
        out_specs=pl.BlockSpec((MB, D), lambda i: (i, 0)),
        out_shape=jax.ShapeDtypeStruct((N, D), jnp.float32),
        compiler_params=pltpu.CompilerParams(
            dimension_semantics=("arbitrary",)),
    )(xf, xf, w4, b)


def _merge_body(part, xb0, xb1, w4, bat, g_ref, out_ref):
    i = pl.program_id(0)
    ho = (part[...]
          + jnp.dot(xb0[...], w4[2], preferred_element_type=jnp.float32)
          + jnp.dot(xb1[...], w4[3], preferred_element_type=jnp.float32))
    out_ref[...] = ho

    @pl.when(i == 0)
    def _():
        g_ref[...] = jnp.full((NUM_GRAPHS, D), -jnp.inf, jnp.float32)

    batv = bat[...]
    for g in range(NUM_GRAPHS):
        m = jnp.where(batv == g, ho, -jnp.inf)
        part = jnp.max(m, axis=0, keepdims=True)
        g_ref[g:g + 1, :] = jnp.maximum(g_ref[g:g + 1, :], part)


def _merge_call(part, xb, w4, batb):
    return pl.pallas_call(
        _merge_body,
        grid=(GRID,),
        in_specs=[
            pl.BlockSpec((MB, D), lambda i: (i, 0)),
            pl.BlockSpec((MB, HALF), lambda i: (i, 0)),
            pl.BlockSpec((MB, HALF), lambda i: (i + GRID, 0)),
            pl.BlockSpec((4, HALF, D), lambda i: (0, 0, 0)),
            pl.BlockSpec((MB, D), lambda i: (i, 0)),
        ],
        out_specs=[
            pl.BlockSpec((NUM_GRAPHS, D), lambda i: (0, 0)),
            pl.BlockSpec((MB, D), lambda i: (i, 0)),
        ],
        out_shape=[
            jax.ShapeDtypeStruct((NUM_GRAPHS, D), jnp.float32),
            jax.ShapeDtypeStruct((N, D), jnp.float32),
        ],
        compiler_params=pltpu.CompilerParams(
            dimension_semantics=("arbitrary",)),
    )(part, xb, xb, w4, batb)


def _prep_edges(ei):
    src = ei[0]
    dst = ei[1]
    pad = E_PAD - E
    dstp = jnp.concatenate([dst, jnp.zeros((pad,), jnp.int32)])
    srcp = jnp.concatenate([src, jnp.full((pad,), N, jnp.int32)])
    dst2 = jnp.stack([dstp, dstp + N]).reshape(2 * NS * EPT)
    src2 = srcp.reshape(NS, CH, CHUNK)
    return dst2, src2


def kernel(h, fw_edge_index, bw_edge_index, batch, W_merge, b_merge):
    chain = _make_chain()
    x0 = h.reshape(N, 2, HALF).transpose(1, 0, 2).reshape(2 * N, HALF)
    fw_dst, fw_src = _prep_edges(fw_edge_index)
    bw_dst, bw_src = _prep_edges(bw_edge_index)
    w4 = W_merge.reshape(D, 4, HALF).transpose(1, 2, 0)
    batb = jnp.broadcast_to(batch[:, None], (N, D))
    xf = chain(x0, fw_dst, fw_src)[2]
    part = _fw_merge_call(xf, w4, b_merge.reshape(1, D))
    xb = chain(x0, bw_dst, bw_src)[2]
    g_h, h_out = _merge_call(part, xb, w4, batb)
    return (g_h, h_out)
